# Initial kernel scaffold; baseline (speedup 1.0000x reference)
import jax
import jax.numpy as jnp
from jax.experimental import pallas as pl


def kernel(x, edge_index, batch, W0, b0, gn0_w, gn0_b, gn0_ms, W1, b1, gn1_w, gn1_b, gn1_ms, W2, b2, gn2_w, gn2_b, gn2_ms):
    raise NotImplementedError("write your pallas kernel here")


# speedup vs baseline: 7.8653x; 7.8653x over previous
import functools

import jax
import jax.numpy as jnp
from jax import lax
from jax.experimental import pallas as pl
from jax.experimental.pallas import tpu as pltpu
from jax.experimental.pallas import tpu_sc as plsc

N_NODES = 10000
N_EDGES = 320000
G = 64
D = 128
EPS = 1e-5

NPAD = 10112
CH = 128
NCHUNK = 79
EPT = NCHUNK * CH
NSC = 2
NTEC = 16
EPAD = EPT * NSC * NTEC
STRIPE = NPAD // NTEC


@functools.lru_cache(maxsize=1)
def _sc_kernels():
    mesh = plsc.VectorSubcoreMesh(core_axis_name="c", subcore_axis_name="s")

    @functools.partial(
        pl.kernel,
        out_type=jax.ShapeDtypeStruct((NSC, NPAD, 16), jnp.float32),
        mesh=mesh,
        compiler_params=pltpu.CompilerParams(use_tc_tiling_on_sc=False),
        scratch_types=[
            pltpu.VMEM((CH,), jnp.int32),
            pltpu.VMEM((CH, 16), jnp.float32),
            pltpu.VMEM_SHARED((NPAD, 16), jnp.float32),
        ],
    )
    def sc_degree(dst_hbm, zeros_hbm, ones_hbm, out_hbm, dst_v, ones_v, acc_sh):
        c = lax.axis_index("c")
        s = lax.axis_index("s")
        pltpu.sync_copy(ones_hbm, ones_v)
        pltpu.sync_copy(zeros_hbm.at[pl.ds(s * STRIPE, STRIPE)],
                        acc_sh.at[pl.ds(s * STRIPE, STRIPE)])
        plsc.subcore_barrier()
        base = (s * NSC + c) * EPT

        def body(j, carry):
            pltpu.sync_copy(dst_hbm.at[pl.ds(base + j * CH, CH)], dst_v)
            pltpu.sync_copy(ones_v, acc_sh.at[dst_v], add=True)
            return carry

        lax.fori_loop(0, NCHUNK, body, 0)
        plsc.subcore_barrier()
        pltpu.sync_copy(acc_sh.at[pl.ds(s * STRIPE, STRIPE)],
                        out_hbm.at[c].at[pl.ds(s * STRIPE, STRIPE)])

    @functools.partial(
        pl.kernel,
        out_type=jax.ShapeDtypeStruct((NSC, NPAD, D), jnp.float32),
        mesh=mesh,
        scratch_types=[
            pltpu.VMEM((CH,), jnp.int32),
            pltpu.VMEM((CH,), jnp.int32),
            pltpu.VMEM((CH, D), jnp.float32),
            pltpu.VMEM_SHARED((NPAD, D), jnp.float32),
            pltpu.SemaphoreType.DMA,
        ],
    )
    def sc_scatter(g_hbm, src_hbm, dst_hbm, zeros_hbm, out_hbm,
                   src_v, dst_v, rows_v, acc_sh, sem):
        c = lax.axis_index("c")
        s = lax.axis_index("s")
        pltpu.sync_copy(zeros_hbm.at[pl.ds(s * STRIPE, STRIPE)],
                        acc_sh.at[pl.ds(s * STRIPE, STRIPE)])
        plsc.subcore_barrier()
        base = (s * NSC + c) * EPT

        def body(j, carry):
            off = base + j * CH
            pltpu.sync_copy(src_hbm.at[pl.ds(off, CH)], src_v)
            pltpu.sync_copy(dst_hbm.at[pl.ds(off, CH)], dst_v)
            pltpu.async_copy(g_hbm.at[src_v], rows_v, sem).wait()
            pltpu.sync_copy(rows_v, acc_sh.at[dst_v], add=True)
            return carry

        lax.fori_loop(0, NCHUNK, body, 0)
        plsc.subcore_barrier()
        pltpu.sync_copy(acc_sh.at[pl.ds(s * STRIPE, STRIPE)],
                        out_hbm.at[c].at[pl.ds(s * STRIPE, STRIPE)])

    return sc_degree, sc_scatter


def _b0_body(x_ref, w_ref, dega_ref, degb_ref, dis_ref, g_ref):
    deg = dega_ref[:, 0:1] + degb_ref[:, 0:1] + 1.0
    dis = lax.rsqrt(deg)
    hw = jnp.dot(x_ref[...], w_ref[...], preferred_element_type=jnp.float32,
                 precision=lax.Precision.HIGHEST)
    dis_ref[...] = dis
    g_ref[...] = hw * dis


_b0_call = pl.pallas_call(
    _b0_body,
    out_shape=[jax.ShapeDtypeStruct((NPAD, 1), jnp.float32),
               jax.ShapeDtypeStruct((NPAD, D), jnp.float32)],
)


def _oh_mat(brow):
    return (lax.broadcasted_iota(jnp.int32, (G, NPAD), 0)
            == brow).astype(jnp.float32)


def _oht_mat(bcol):
    return (lax.broadcasted_iota(jnp.int32, (NPAD, G), 1)
            == bcol).astype(jnp.float32)


def _stats_body(acc0_ref, acc1_ref, g_ref, dis_ref, b_ref, brow_ref,
                conv_ref, mean_ref, cnt_ref):
    conv = dis_ref[...] * (acc0_ref[...] + acc1_ref[...] + g_ref[...]) + b_ref[...]
    oh = _oh_mat(brow_ref[...])
    cnt = jnp.maximum(jnp.sum(oh, axis=1, keepdims=True), 1.0)
    mean_ref[...] = jnp.dot(oh, conv, preferred_element_type=jnp.float32,
                            precision=lax.Precision.HIGHEST) / cnt
    conv_ref[...] = conv
    cnt_ref[...] = cnt


_stats_call = pl.pallas_call(
    _stats_body,
    out_shape=[jax.ShapeDtypeStruct((NPAD, D), jnp.float32),
               jax.ShapeDtypeStruct((G, D), jnp.float32),
               jax.ShapeDtypeStruct((G, 1), jnp.float32)],
)


def _norm_relu(conv_ref, mean_ref, cnt_ref, gw_ref, gb_ref, gms_ref,
               brow_ref, bcol_ref):
    oht = _oht_mat(bcol_ref[...])
    outc = conv_ref[...] - jnp.dot(
        oht, mean_ref[...], preferred_element_type=jnp.float32,
        precision=lax.Precision.HIGHEST) * gms_ref[...]
    oh = _oh_mat(brow_ref[...])
    var = jnp.dot(oh, outc * outc, preferred_element_type=jnp.float32,
                  precision=lax.Precision.HIGHEST) / cnt_ref[...]
    r = gw_ref[...] / jnp.sqrt(var + EPS)
    rb = jnp.dot(oht, r, preferred_element_type=jnp.float32,
                 precision=lax.Precision.HIGHEST)
    return jnp.maximum(outc * rb + gb_ref[...], 0.0), oh


def _apply_mid_body(conv_ref, mean_ref, cnt_ref, gw_ref, gb_ref, gms_ref,
                    brow_ref, bcol_ref, dis_ref, wn_ref, gnext_ref):
    hn, _ = _norm_relu(conv_ref, mean_ref, cnt_ref, gw_ref, gb_ref, gms_ref,
                       brow_ref, bcol_ref)
    gnext_ref[...] = dis_ref[...] * jnp.dot(
        hn, wn_ref[...], preferred_element_type=jnp.float32,
        precision=lax.Precision.HIGHEST)


_apply_mid_call = pl.pallas_call(
    _apply_mid_body,
    out_shape=[jax.ShapeDtypeStruct((NPAD, D), jnp.float32)],
)


def _apply_last_body(conv_ref, mean_ref, cnt_ref, gw_ref, gb_ref, gms_ref,
                     brow_ref, bcol_ref, pmean_ref, pmax_ref):
    hn, oh = _norm_relu(conv_ref, mean_ref, cnt_ref, gw_ref, gb_ref, gms_ref,
                        brow_ref, bcol_ref)
    pmean_ref[...] = jnp.dot(oh, hn, preferred_element_type=jnp.float32,
                             precision=lax.Precision.HIGHEST) / cnt_ref[...]

    def body(gi, carry):
        m = jnp.where(bcol_ref[...] == gi, hn, -jnp.inf)
        mx = jnp.max(m, axis=0, keepdims=True)
        pmax_ref[pl.ds(gi, 1), :] = jnp.where(jnp.isfinite(mx), mx, 0.0)
        return carry

    lax.fori_loop(0, G, body, 0)


_apply_last_call = pl.pallas_call(
    _apply_last_body,
    out_shape=[jax.ShapeDtypeStruct((G, D), jnp.float32),
               jax.ShapeDtypeStruct((G, D), jnp.float32)],
)


def kernel(x, edge_index, batch, W0, b0, gn0_w, gn0_b, gn0_ms,
           W1, b1, gn1_w, gn1_b, gn1_ms, W2, b2, gn2_w, gn2_b, gn2_ms):
    pad_e = EPAD - N_EDGES
    srcp = jnp.concatenate(
        [edge_index[0], jnp.full((pad_e,), N_NODES, jnp.int32)])
    dstp = jnp.concatenate(
        [edge_index[1], jnp.full((pad_e,), N_NODES, jnp.int32)])
    xp = jnp.pad(x, ((0, NPAD - N_NODES), (0, 0)))
    batchp = jnp.pad(batch, (0, NPAD - N_NODES), constant_values=G)
    brow = batchp.reshape(1, NPAD)
    bcol = batchp.reshape(NPAD, 1)
    zeros128 = jnp.zeros((NPAD, D), jnp.float32)
    zeros16 = jnp.zeros((NPAD, 16), jnp.float32)
    ones16 = jnp.ones((CH, 16), jnp.float32)

    sc_degree, sc_scatter = _sc_kernels()
    degs = sc_degree(dstp, zeros16, ones16)
    dis, g = _b0_call(xp, W0, degs[0], degs[1])

    params = [(b0, gn0_w, gn0_b, gn0_ms), (b1, gn1_w, gn1_b, gn1_ms),
              (b2, gn2_w, gn2_b, gn2_ms)]
    wnext = [W1, W2]
    for l in range(3):
        bb, gw, gb, gms = (p.reshape(1, D) for p in params[l])
        accs = sc_scatter(g, srcp, dstp, zeros128)
        conv, mean, cnt = _stats_call(accs[0], accs[1], g, dis, bb, brow)
        if l < 2:
            (g,) = _apply_mid_call(conv, mean, cnt, gw, gb, gms,
                                   brow, bcol, dis, wnext[l])
        else:
            x_mean, x_max = _apply_last_call(conv, mean, cnt, gw, gb, gms,
                                             brow, bcol)
    return jnp.concatenate([x_mean, x_max], axis=1)


# speedup vs baseline: 9.5666x; 1.2163x over previous
import functools

import jax
import jax.numpy as jnp
from jax import lax
from jax.experimental import pallas as pl
from jax.experimental.pallas import tpu as pltpu
from jax.experimental.pallas import tpu_sc as plsc

N_NODES = 10000
N_EDGES = 320000
G = 64
D = 128
EPS = 1e-5

NPAD = 10112
CH = 128
NSC = 2
NTEC = 16
DH = D // NSC
DEG_NCHUNK = 80
DEG_EPT = DEG_NCHUNK * CH
EPAD = DEG_EPT * NSC * NTEC
NCHUNK = EPAD // (NTEC * CH)
NB = 4
NGRP = NCHUNK // NB
STRIPE = NPAD // NTEC


@functools.lru_cache(maxsize=1)
def _sc_kernels():
    mesh = plsc.VectorSubcoreMesh(core_axis_name="c", subcore_axis_name="s")

    @functools.partial(
        pl.kernel,
        out_type=jax.ShapeDtypeStruct((NSC, NPAD, 16), jnp.float32),
        mesh=mesh,
        compiler_params=pltpu.CompilerParams(use_tc_tiling_on_sc=False),
        scratch_types=[
            pltpu.VMEM((CH,), jnp.int32),
            pltpu.VMEM((CH, 16), jnp.float32),
            pltpu.VMEM_SHARED((NPAD, 16), jnp.float32),
        ],
    )
    def sc_degree(dst_hbm, zeros_hbm, ones_hbm, out_hbm, dst_v, ones_v, acc_sh):
        c = lax.axis_index("c")
        s = lax.axis_index("s")
        pltpu.sync_copy(ones_hbm, ones_v)
        pltpu.sync_copy(zeros_hbm.at[pl.ds(s * STRIPE, STRIPE)],
                        acc_sh.at[pl.ds(s * STRIPE, STRIPE)])
        plsc.subcore_barrier()
        base = (s * NSC + c) * DEG_EPT

        def body(j, carry):
            pltpu.sync_copy(dst_hbm.at[pl.ds(base + j * CH, CH)], dst_v)
            pltpu.sync_copy(ones_v, acc_sh.at[dst_v], add=True)
            return carry

        lax.fori_loop(0, DEG_NCHUNK, body, 0)
        plsc.subcore_barrier()
        pltpu.sync_copy(acc_sh.at[pl.ds(s * STRIPE, STRIPE)],
                        out_hbm.at[c].at[pl.ds(s * STRIPE, STRIPE)])

    @functools.partial(
        pl.kernel,
        out_type=jax.ShapeDtypeStruct((NSC, NPAD, DH), jnp.float32),
        mesh=mesh,
        compiler_params=pltpu.CompilerParams(use_tc_tiling_on_sc=False),
        scratch_types=[
            pltpu.VMEM((NCHUNK, CH), jnp.int32),
            pltpu.VMEM((NCHUNK, CH), jnp.int32),
            pltpu.VMEM((NB, CH, DH), jnp.float32),
            pltpu.VMEM_SHARED((NPAD, DH), jnp.float32),
            pltpu.SemaphoreType.DMA((NB,)),
            pltpu.SemaphoreType.DMA((NB,)),
        ],
    )
    def sc_scatter(g2_hbm, src3_hbm, dst3_hbm, zeros_hbm, out_hbm,
                   src_all, dst_all, rows, acc_sh, gsem, ssem):
        c = lax.axis_index("c")
        s = lax.axis_index("s")
        gt = g2_hbm.at[c]
        pltpu.sync_copy(src3_hbm.at[s], src_all)
        pltpu.sync_copy(dst3_hbm.at[s], dst_all)
        pltpu.sync_copy(zeros_hbm.at[pl.ds(s * STRIPE, STRIPE)],
                        acc_sh.at[pl.ds(s * STRIPE, STRIPE)])
        for b in range(NB):
            pltpu.async_copy(gt.at[src_all.at[b]], rows.at[b], gsem.at[b])
        plsc.subcore_barrier()

        def grp(t, carry):
            for b in range(NB):
                j = t * NB + b
                pltpu.make_async_copy(
                    gt.at[src_all.at[j]], rows.at[b], gsem.at[b]).wait()
                pltpu.async_copy(
                    rows.at[b], acc_sh.at[dst_all.at[j]], ssem.at[b], add=True)
            for b in range(NB):
                j = t * NB + b
                pltpu.make_async_copy(
                    rows.at[b], acc_sh.at[dst_all.at[j]], ssem.at[b]).wait()

                @pl.when(t < NGRP - 1)
                def _issue_next(b=b, t=t):
                    jn = (t + 1) * NB + b
                    pltpu.async_copy(
                        gt.at[src_all.at[jn]], rows.at[b], gsem.at[b])

            return carry

        lax.fori_loop(0, NGRP, grp, 0)
        plsc.subcore_barrier()
        pltpu.sync_copy(acc_sh.at[pl.ds(s * STRIPE, STRIPE)],
                        out_hbm.at[c].at[pl.ds(s * STRIPE, STRIPE)])

    return sc_degree, sc_scatter


def _b0_body(x_ref, w_ref, dega_ref, degb_ref, dis_ref, g_ref):
    deg = dega_ref[:, 0:1] + degb_ref[:, 0:1] + 1.0
    dis = lax.rsqrt(deg)
    hw = jnp.dot(x_ref[...], w_ref[...], preferred_element_type=jnp.float32,
                 precision=lax.Precision.HIGHEST)
    g = hw * dis
    dis_ref[...] = dis
    g_ref[0] = g[:, :DH]
    g_ref[1] = g[:, DH:]


_b0_call = pl.pallas_call(
    _b0_body,
    out_shape=[jax.ShapeDtypeStruct((NPAD, 1), jnp.float32),
               jax.ShapeDtypeStruct((NSC, NPAD, DH), jnp.float32)],
)


def _oh_mat(brow):
    return (lax.broadcasted_iota(jnp.int32, (G, NPAD), 0)
            == brow).astype(jnp.float32)


def _oht_mat(bcol):
    return (lax.broadcasted_iota(jnp.int32, (NPAD, G), 1)
            == bcol).astype(jnp.float32)


def _stats_body(acc_ref, g_ref, dis_ref, b_ref, brow_ref,
                conv_ref, mean_ref, cnt_ref):
    acc = jnp.concatenate([acc_ref[0], acc_ref[1]], axis=1)
    g = jnp.concatenate([g_ref[0], g_ref[1]], axis=1)
    conv = dis_ref[...] * (acc + g) + b_ref[...]
    oh = _oh_mat(brow_ref[...])
    cnt = jnp.maximum(jnp.sum(oh, axis=1, keepdims=True), 1.0)
    mean_ref[...] = jnp.dot(oh, conv, preferred_element_type=jnp.float32,
                            precision=lax.Precision.HIGHEST) / cnt
    conv_ref[...] = conv
    cnt_ref[...] = cnt


_stats_call = pl.pallas_call(
    _stats_body,
    out_shape=[jax.ShapeDtypeStruct((NPAD, D), jnp.float32),
               jax.ShapeDtypeStruct((G, D), jnp.float32),
               jax.ShapeDtypeStruct((G, 1), jnp.float32)],
)


def _norm_relu(conv_ref, mean_ref, cnt_ref, gw_ref, gb_ref, gms_ref,
               brow_ref, bcol_ref):
    oht = _oht_mat(bcol_ref[...])
    outc = conv_ref[...] - jnp.dot(
        oht, mean_ref[...], preferred_element_type=jnp.float32,
        precision=lax.Precision.HIGHEST) * gms_ref[...]
    oh = _oh_mat(brow_ref[...])
    var = jnp.dot(oh, outc * outc, preferred_element_type=jnp.float32,
                  precision=lax.Precision.HIGHEST) / cnt_ref[...]
    r = gw_ref[...] / jnp.sqrt(var + EPS)
    rb = jnp.dot(oht, r, preferred_element_type=jnp.float32,
                 precision=lax.Precision.HIGHEST)
    return jnp.maximum(outc * rb + gb_ref[...], 0.0), oh


def _apply_mid_body(conv_ref, mean_ref, cnt_ref, gw_ref, gb_ref, gms_ref,
                    brow_ref, bcol_ref, dis_ref, wn_ref, gnext_ref):
    hn, _ = _norm_relu(conv_ref, mean_ref, cnt_ref, gw_ref, gb_ref, gms_ref,
                       brow_ref, bcol_ref)
    gnext = dis_ref[...] * jnp.dot(
        hn, wn_ref[...], preferred_element_type=jnp.float32,
        precision=lax.Precision.HIGHEST)
    gnext_ref[0] = gnext[:, :DH]
    gnext_ref[1] = gnext[:, DH:]


_apply_mid_call = pl.pallas_call(
    _apply_mid_body,
    out_shape=[jax.ShapeDtypeStruct((NSC, NPAD, DH), jnp.float32)],
)


def _apply_last_body(conv_ref, mean_ref, cnt_ref, gw_ref, gb_ref, gms_ref,
                     brow_ref, bcol_ref, pmean_ref, pmax_ref):
    hn, oh = _norm_relu(conv_ref, mean_ref, cnt_ref, gw_ref, gb_ref, gms_ref,
                        brow_ref, bcol_ref)
    pmean_ref[...] = jnp.dot(oh, hn, preferred_element_type=jnp.float32,
                             precision=lax.Precision.HIGHEST) / cnt_ref[...]

    def body(gi, carry):
        m = jnp.where(bcol_ref[...] == gi, hn, -jnp.inf)
        mx = jnp.max(m, axis=0, keepdims=True)
        pmax_ref[pl.ds(gi, 1), :] = jnp.where(jnp.isfinite(mx), mx, 0.0)
        return carry

    lax.fori_loop(0, G, body, 0)


_apply_last_call = pl.pallas_call(
    _apply_last_body,
    out_shape=[jax.ShapeDtypeStruct((G, D), jnp.float32),
               jax.ShapeDtypeStruct((G, D), jnp.float32)],
)


def kernel(x, edge_index, batch, W0, b0, gn0_w, gn0_b, gn0_ms,
           W1, b1, gn1_w, gn1_b, gn1_ms, W2, b2, gn2_w, gn2_b, gn2_ms):
    pad_e = EPAD - N_EDGES
    srcp = jnp.concatenate(
        [edge_index[0], jnp.full((pad_e,), N_NODES, jnp.int32)])
    dstp = jnp.concatenate(
        [edge_index[1], jnp.full((pad_e,), N_NODES, jnp.int32)])
    src3 = srcp.reshape(NTEC, NCHUNK, CH)
    dst3 = dstp.reshape(NTEC, NCHUNK, CH)
    xp = jnp.pad(x, ((0, NPAD - N_NODES), (0, 0)))
    batchp = jnp.pad(batch, (0, NPAD - N_NODES), constant_values=G)
    brow = batchp.reshape(1, NPAD)
    bcol = batchp.reshape(NPAD, 1)
    zeros64 = jnp.zeros((NPAD, DH), jnp.float32)
    zeros16 = jnp.zeros((NPAD, 16), jnp.float32)
    ones16 = jnp.ones((CH, 16), jnp.float32)

    sc_degree, sc_scatter = _sc_kernels()
    degs = sc_degree(dstp, zeros16, ones16)
    dis, g = _b0_call(xp, W0, degs[0], degs[1])

    params = [(b0, gn0_w, gn0_b, gn0_ms), (b1, gn1_w, gn1_b, gn1_ms),
              (b2, gn2_w, gn2_b, gn2_ms)]
    wnext = [W1, W2]
    for l in range(3):
        bb, gw, gb, gms = (p.reshape(1, D) for p in params[l])
        accs = sc_scatter(g, src3, dst3, zeros64)
        conv, mean, cnt = _stats_call(accs, g, dis, bb, brow)
        if l < 2:
            (g,) = _apply_mid_call(conv, mean, cnt, gw, gb, gms,
                                   brow, bcol, dis, wnext[l])
        else:
            x_mean, x_max = _apply_last_call(conv, mean, cnt, gw, gb, gms,
                                             brow, bcol)
    return jnp.concatenate([x_mean, x_max], axis=1)


# speedup vs baseline: 10.1476x; 1.0607x over previous
import functools

import jax
import jax.numpy as jnp
from jax import lax
from jax.experimental import pallas as pl
from jax.experimental.pallas import tpu as pltpu
from jax.experimental.pallas import tpu_sc as plsc

N_NODES = 10000
N_EDGES = 320000
G = 64
D = 128
EPS = 1e-5

NPAD = 10112
CH = 128
NSC = 2
NTEC = 16
DH = D // NSC
DEG_NCHUNK = 80
DEG_EPT = DEG_NCHUNK * CH
EPAD = DEG_EPT * NSC * NTEC
NCHUNK = EPAD // (NTEC * CH)
NB = 5
NGRP = NCHUNK // NB
STRIPE = NPAD // NTEC


@functools.lru_cache(maxsize=1)
def _sc_kernels():
    mesh = plsc.VectorSubcoreMesh(core_axis_name="c", subcore_axis_name="s")

    @functools.partial(
        pl.kernel,
        out_type=jax.ShapeDtypeStruct((NSC, NPAD, 16), jnp.float32),
        mesh=mesh,
        compiler_params=pltpu.CompilerParams(use_tc_tiling_on_sc=False),
        scratch_types=[
            pltpu.VMEM((CH,), jnp.int32),
            pltpu.VMEM((CH, 16), jnp.float32),
            pltpu.VMEM_SHARED((NPAD, 16), jnp.float32),
        ],
    )
    def sc_degree(dst_hbm, zeros_hbm, ones_hbm, out_hbm, dst_v, ones_v, acc_sh):
        c = lax.axis_index("c")
        s = lax.axis_index("s")
        pltpu.sync_copy(ones_hbm, ones_v)
        pltpu.sync_copy(zeros_hbm.at[pl.ds(s * STRIPE, STRIPE)],
                        acc_sh.at[pl.ds(s * STRIPE, STRIPE)])
        plsc.subcore_barrier()
        base = (s * NSC + c) * DEG_EPT

        def body(j, carry):
            pltpu.sync_copy(dst_hbm.at[pl.ds(base + j * CH, CH)], dst_v)
            pltpu.sync_copy(ones_v, acc_sh.at[dst_v], add=True)
            return carry

        lax.fori_loop(0, DEG_NCHUNK, body, 0)
        plsc.subcore_barrier()
        pltpu.sync_copy(acc_sh.at[pl.ds(s * STRIPE, STRIPE)],
                        out_hbm.at[c].at[pl.ds(s * STRIPE, STRIPE)])

    @functools.partial(
        pl.kernel,
        out_type=jax.ShapeDtypeStruct((NSC, NPAD, DH), jnp.float32),
        mesh=mesh,
        compiler_params=pltpu.CompilerParams(use_tc_tiling_on_sc=False),
        scratch_types=[
            pltpu.VMEM((NCHUNK, CH), jnp.int32),
            pltpu.VMEM((NCHUNK, CH), jnp.int32),
            pltpu.VMEM((NB, CH, DH), jnp.float32),
            pltpu.VMEM_SHARED((NPAD, DH), jnp.float32),
            pltpu.SemaphoreType.DMA((NB,)),
            pltpu.SemaphoreType.DMA((NB,)),
        ],
    )
    def sc_scatter(g2_hbm, src3_hbm, dst3_hbm, zeros_hbm, out_hbm,
                   src_all, dst_all, rows, acc_sh, gsem, ssem):
        c = lax.axis_index("c")
        s = lax.axis_index("s")
        gt = g2_hbm.at[c]
        pltpu.sync_copy(src3_hbm.at[s], src_all)
        pltpu.sync_copy(dst3_hbm.at[s], dst_all)
        pltpu.sync_copy(zeros_hbm.at[pl.ds(s * STRIPE, STRIPE)],
                        acc_sh.at[pl.ds(s * STRIPE, STRIPE)])
        for b in range(NB):
            pltpu.async_copy(gt.at[src_all.at[b]], rows.at[b], gsem.at[b])
        plsc.subcore_barrier()

        def grp(t, carry):
            for b in range(NB):
                j = t * NB + b
                pltpu.make_async_copy(
                    gt.at[src_all.at[j]], rows.at[b], gsem.at[b]).wait()
                pltpu.async_copy(
                    rows.at[b], acc_sh.at[dst_all.at[j]], ssem.at[b], add=True)
            for b in range(NB):
                j = t * NB + b
                pltpu.make_async_copy(
                    rows.at[b], acc_sh.at[dst_all.at[j]], ssem.at[b]).wait()

                @pl.when(t < NGRP - 1)
                def _issue_next(b=b, t=t):
                    jn = (t + 1) * NB + b
                    pltpu.async_copy(
                        gt.at[src_all.at[jn]], rows.at[b], gsem.at[b])

            return carry

        lax.fori_loop(0, NGRP, grp, 0)
        plsc.subcore_barrier()
        pltpu.sync_copy(acc_sh.at[pl.ds(s * STRIPE, STRIPE)],
                        out_hbm.at[c].at[pl.ds(s * STRIPE, STRIPE)])

    return sc_degree, sc_scatter


def _b0_body(x_ref, w_ref, dega_ref, degb_ref, dis_ref, g_ref):
    deg = dega_ref[:, 0:1] + degb_ref[:, 0:1] + 1.0
    dis = lax.rsqrt(deg)
    hw = jnp.dot(x_ref[...], w_ref[...], preferred_element_type=jnp.float32,
                 precision=lax.Precision.HIGHEST)
    g = hw * dis
    dis_ref[...] = dis
    g_ref[0] = g[:, :DH]
    g_ref[1] = g[:, DH:]


_b0_call = pl.pallas_call(
    _b0_body,
    out_shape=[jax.ShapeDtypeStruct((NPAD, 1), jnp.float32),
               jax.ShapeDtypeStruct((NSC, NPAD, DH), jnp.float32)],
)


def _oh_mat(brow):
    return (lax.broadcasted_iota(jnp.int32, (G, NPAD), 0)
            == brow).astype(jnp.float32)


def _oht_mat(bcol):
    return (lax.broadcasted_iota(jnp.int32, (NPAD, G), 1)
            == bcol).astype(jnp.float32)


def _stats_body(acc_ref, g_ref, dis_ref, b_ref, brow_ref,
                conv_ref, mean_ref, cnt_ref):
    acc = jnp.concatenate([acc_ref[0], acc_ref[1]], axis=1)
    g = jnp.concatenate([g_ref[0], g_ref[1]], axis=1)
    conv = dis_ref[...] * (acc + g) + b_ref[...]
    oh = _oh_mat(brow_ref[...])
    cnt = jnp.maximum(jnp.sum(oh, axis=1, keepdims=True), 1.0)
    mean_ref[...] = jnp.dot(oh, conv, preferred_element_type=jnp.float32,
                            precision=lax.Precision.HIGHEST) / cnt
    conv_ref[...] = conv
    cnt_ref[...] = cnt


_stats_call = pl.pallas_call(
    _stats_body,
    out_shape=[jax.ShapeDtypeStruct((NPAD, D), jnp.float32),
               jax.ShapeDtypeStruct((G, D), jnp.float32),
               jax.ShapeDtypeStruct((G, 1), jnp.float32)],
)


def _norm_relu(conv_ref, mean_ref, cnt_ref, gw_ref, gb_ref, gms_ref,
               brow_ref, bcol_ref):
    oht = _oht_mat(bcol_ref[...])
    outc = conv_ref[...] - jnp.dot(
        oht, mean_ref[...], preferred_element_type=jnp.float32,
        precision=lax.Precision.HIGHEST) * gms_ref[...]
    oh = _oh_mat(brow_ref[...])
    var = jnp.dot(oh, outc * outc, preferred_element_type=jnp.float32,
                  precision=lax.Precision.HIGHEST) / cnt_ref[...]
    r = gw_ref[...] / jnp.sqrt(var + EPS)
    rb = jnp.dot(oht, r, preferred_element_type=jnp.float32,
                 precision=lax.Precision.HIGHEST)
    return jnp.maximum(outc * rb + gb_ref[...], 0.0), oh


def _apply_mid_body(conv_ref, mean_ref, cnt_ref, gw_ref, gb_ref, gms_ref,
                    brow_ref, bcol_ref, dis_ref, wn_ref, gnext_ref):
    hn, _ = _norm_relu(conv_ref, mean_ref, cnt_ref, gw_ref, gb_ref, gms_ref,
                       brow_ref, bcol_ref)
    gnext = dis_ref[...] * jnp.dot(
        hn, wn_ref[...], preferred_element_type=jnp.float32,
        precision=lax.Precision.HIGHEST)
    gnext_ref[0] = gnext[:, :DH]
    gnext_ref[1] = gnext[:, DH:]


_apply_mid_call = pl.pallas_call(
    _apply_mid_body,
    out_shape=[jax.ShapeDtypeStruct((NSC, NPAD, DH), jnp.float32)],
)


NBLK = NPAD // 8


def _apply_last_body(conv_ref, mean_ref, cnt_ref, gw_ref, gb_ref, gms_ref,
                     brow_ref, bcol_ref, se_ref, pmean_ref, pmax_ref, hn_ref):
    hn, oh = _norm_relu(conv_ref, mean_ref, cnt_ref, gw_ref, gb_ref, gms_ref,
                        brow_ref, bcol_ref)
    pmean_ref[...] = jnp.dot(oh, hn, preferred_element_type=jnp.float32,
                             precision=lax.Precision.HIGHEST) / cnt_ref[...]
    hn_ref[...] = hn
    bmax = jnp.max(hn.reshape(NBLK, 8, D), axis=1)
    blk8 = lax.broadcasted_iota(jnp.int32, (NBLK, 1), 0) * 8

    def body(gi, carry):
        a = se_ref[0, gi]
        b = se_ref[1, gi]
        full = (blk8 >= a) & ((blk8 + 8) <= b)
        inner = jnp.max(jnp.where(full, bmax, -jnp.inf), axis=0, keepdims=True)
        sb = a // 8 * 8
        eb = jnp.maximum(b - 1, 0) // 8 * 8
        ra = hn_ref[pl.ds(sb, 8), :]
        ca = bcol_ref[pl.ds(sb, 8), :]
        ma = jnp.max(jnp.where(ca == gi, ra, -jnp.inf), axis=0, keepdims=True)
        rb = hn_ref[pl.ds(eb, 8), :]
        cb = bcol_ref[pl.ds(eb, 8), :]
        mb = jnp.max(jnp.where(cb == gi, rb, -jnp.inf), axis=0, keepdims=True)
        mx = jnp.maximum(inner, jnp.maximum(ma, mb))
        pmax_ref[pl.ds(gi, 1), :] = jnp.where(jnp.isfinite(mx), mx, 0.0)
        return carry

    lax.fori_loop(0, G, body, 0)


_apply_last_call = pl.pallas_call(
    _apply_last_body,
    in_specs=[pl.BlockSpec(), pl.BlockSpec(), pl.BlockSpec(), pl.BlockSpec(),
              pl.BlockSpec(), pl.BlockSpec(), pl.BlockSpec(), pl.BlockSpec(),
              pl.BlockSpec(memory_space=pltpu.SMEM)],
    out_shape=[jax.ShapeDtypeStruct((G, D), jnp.float32),
               jax.ShapeDtypeStruct((G, D), jnp.float32)],
    scratch_shapes=[pltpu.VMEM((NPAD, D), jnp.float32)],
)


def kernel(x, edge_index, batch, W0, b0, gn0_w, gn0_b, gn0_ms,
           W1, b1, gn1_w, gn1_b, gn1_ms, W2, b2, gn2_w, gn2_b, gn2_ms):
    pad_e = EPAD - N_EDGES
    srcp = jnp.concatenate(
        [edge_index[0], jnp.full((pad_e,), N_NODES, jnp.int32)])
    dstp = jnp.concatenate(
        [edge_index[1], jnp.full((pad_e,), N_NODES, jnp.int32)])
    src3 = srcp.reshape(NTEC, NCHUNK, CH)
    dst3 = dstp.reshape(NTEC, NCHUNK, CH)
    xp = jnp.pad(x, ((0, NPAD - N_NODES), (0, 0)))
    batchp = jnp.pad(batch, (0, NPAD - N_NODES), constant_values=G)
    brow = batchp.reshape(1, NPAD)
    bcol = batchp.reshape(NPAD, 1)
    gids = jnp.arange(G, dtype=jnp.int32)
    se = jnp.stack([jnp.searchsorted(batch, gids, side="left"),
                    jnp.searchsorted(batch, gids, side="right")]
                   ).astype(jnp.int32)
    zeros64 = jnp.zeros((NPAD, DH), jnp.float32)
    zeros16 = jnp.zeros((NPAD, 16), jnp.float32)
    ones16 = jnp.ones((CH, 16), jnp.float32)

    sc_degree, sc_scatter = _sc_kernels()
    degs = sc_degree(dstp, zeros16, ones16)
    dis, g = _b0_call(xp, W0, degs[0], degs[1])

    params = [(b0, gn0_w, gn0_b, gn0_ms), (b1, gn1_w, gn1_b, gn1_ms),
              (b2, gn2_w, gn2_b, gn2_ms)]
    wnext = [W1, W2]
    for l in range(3):
        bb, gw, gb, gms = (p.reshape(1, D) for p in params[l])
        accs = sc_scatter(g, src3, dst3, zeros64)
        conv, mean, cnt = _stats_call(accs, g, dis, bb, brow)
        if l < 2:
            (g,) = _apply_mid_call(conv, mean, cnt, gw, gb, gms,
                                   brow, bcol, dis, wnext[l])
        else:
            x_mean, x_max = _apply_last_call(conv, mean, cnt, gw, gb, gms,
                                             brow, bcol, se)
    return jnp.concatenate([x_mean, x_max], axis=1)
